# Initial kernel scaffold; baseline (speedup 1.0000x reference)
import jax
import jax.numpy as jnp
from jax.experimental import pallas as pl


def kernel(node_fts, edge_fts, edges, Wn, We, a_src, a_dst, a_edge):
    raise NotImplementedError("write your pallas kernel here")


# speedup vs baseline: 11.1496x; 11.1496x over previous
import jax
import jax.numpy as jnp
from jax import lax
from jax.experimental import pallas as pl
from jax.experimental.pallas import tpu as pltpu
from jax.experimental.pallas import tpu_sc as plsc

N = 10000
E = 320000
D = 128
DE = 16
H = 4
NP = 10240
NSUB = 16
NCORE = 2
ECHUNK = E // NSUB
EW = E // (NSUB * NCORE)
EB_SM = 2000
EB_AG = 160
EB_EG = 400
ROWS_T = NP // NSUB

_f32 = jnp.float32
_i32 = jnp.int32

_SC_PARAMS = pltpu.CompilerParams(use_tc_tiling_on_sc=False,
                                  needs_layout_passes=False)


def _prep_nodes_body(x_ref, wn_ref, asrc_ref, adst_ref, h_ref, s_ref, d_ref):
    x = x_ref[...]
    for h in range(H):
        hh = jnp.dot(x, wn_ref[h], preferred_element_type=_f32)
        h_ref[h] = hh
        s_ref[h] = jnp.dot(hh, asrc_ref[h], preferred_element_type=_f32)
        d_ref[h] = jnp.dot(hh, adst_ref[h], preferred_element_type=_f32)


def _prep_edges_body(ef_ref, we_ref, ae_ref, el_ref):
    ef = ef_ref[...]
    for h in range(H):
        q = jnp.dot(we_ref[h], ae_ref[h], preferred_element_type=_f32)
        el_ref[h] = jnp.dot(ef, q, preferred_element_type=_f32)


_BLKN = 1024
_BLKE = 16000


def _tc_prep(node_fts, edge_fts, Wn, We, a_edge, a_src, a_dst):
    h_tbl, s_tbl, d_tbl = pl.pallas_call(
        _prep_nodes_body,
        grid=(NP // _BLKN,),
        in_specs=[
            pl.BlockSpec((_BLKN, D), lambda i: (i, 0)),
            pl.BlockSpec((H, D, D), lambda i: (0, 0, 0)),
            pl.BlockSpec((H, D), lambda i: (0, 0)),
            pl.BlockSpec((H, D), lambda i: (0, 0)),
        ],
        out_specs=[
            pl.BlockSpec((H, _BLKN, D), lambda i: (0, i, 0)),
            pl.BlockSpec((H, _BLKN), lambda i: (0, i)),
            pl.BlockSpec((H, _BLKN), lambda i: (0, i)),
        ],
        out_shape=[
            jax.ShapeDtypeStruct((H, NP, D), _f32),
            jax.ShapeDtypeStruct((H, NP), _f32),
            jax.ShapeDtypeStruct((H, NP), _f32),
        ],
    )(node_fts, Wn, a_src, a_dst)

    el = pl.pallas_call(
        _prep_edges_body,
        grid=(E // _BLKE,),
        in_specs=[
            pl.BlockSpec((_BLKE, DE), lambda i: (i, 0)),
            pl.BlockSpec((H, DE, DE), lambda i: (0, 0, 0)),
            pl.BlockSpec((H, DE), lambda i: (0, 0)),
        ],
        out_specs=pl.BlockSpec((H, _BLKE), lambda i: (0, i)),
        out_shape=jax.ShapeDtypeStruct((H, E), _f32),
    )(edge_fts, We, a_edge)
    return h_tbl, s_tbl, d_tbl, el


def _softmax_kernel(src_hbm, dst_hbm, s_hbm, d_hbm, el_hbm,
                    alpha_hbm, stats_hbm, ex_hbm,
                    s_t, d_t, dn_t, srcb, dstb, elb, exb, alb, statsb,
                    slots, comb):
    c = lax.axis_index("c")
    sid = lax.axis_index("s")
    base = sid * ECHUNK
    zero16 = jnp.zeros((16,), _f32)

    for hg in range(2):
        pltpu.sync_copy(s_hbm.at[pl.ds(c * (2 * NP) + hg * NP, NP)],
                        s_t.at[hg])
        pltpu.sync_copy(d_hbm.at[pl.ds(c * (2 * NP) + hg * NP, NP)],
                        d_t.at[hg])

        def zbody(i, _, hg=hg):
            dn_t[hg, pl.ds(i * 16, 16)] = zero16
            return 0
        lax.fori_loop(0, NP // 16, zbody, 0)

    def blockA(b, _):
        off = base + b * EB_SM
        pltpu.sync_copy(src_hbm.at[pl.ds(off, EB_SM)], srcb)
        pltpu.sync_copy(dst_hbm.at[pl.ds(off, EB_SM)], dstb)
        for hg in range(2):
            pltpu.sync_copy(el_hbm.at[pl.ds((2 * c + hg) * E + off, EB_SM)],
                            elb.at[hg])

        def inner(i, _):
            srcv = srcb[pl.ds(i * 16, 16)]
            dstv = dstb[pl.ds(i * 16, 16)]
            for hg in range(2):
                hgv = jnp.full((16,), hg, _i32)
                sv = plsc.load_gather(s_t, [hgv, srcv])
                dv = plsc.load_gather(d_t, [hgv, dstv])
                l = sv + dv + elb[hg, pl.ds(i * 16, 16)]
                l = jnp.where(l >= 0.0, l, 0.2 * l)
                ex = jnp.exp(l)
                exb[hg, pl.ds(i * 16, 16)] = ex
                plsc.addupdate_scatter(dn_t, [hgv, dstv], ex)
            return 0
        lax.fori_loop(0, EB_SM // 16, inner, 0)
        for hg in range(2):
            pltpu.sync_copy(exb.at[hg],
                            ex_hbm.at[pl.ds((2 * c + hg) * E + off, EB_SM)])
        return 0
    lax.fori_loop(0, ECHUNK // EB_SM, blockA, 0)

    pltpu.sync_copy(dn_t, slots.at[sid])
    plsc.subcore_barrier()
    hg_own = sid // 8
    col0 = (sid % 8) * (NP // 8)
    redbuf = alb.at[0]
    reacc = alb.at[1]

    def zr(i, _):
        reacc[pl.ds(i * 16, 16)] = zero16
        return 0
    lax.fori_loop(0, (NP // 8) // 16, zr, 0)
    for j in range(NSUB):
        pltpu.sync_copy(slots.at[j, hg_own, pl.ds(col0, NP // 8)],
                        redbuf.at[pl.ds(0, NP // 8)])

        def addr(i, _):
            reacc[pl.ds(i * 16, 16)] = (reacc[pl.ds(i * 16, 16)]
                                        + redbuf[pl.ds(i * 16, 16)])
            return 0
        lax.fori_loop(0, (NP // 8) // 16, addr, 0)
    pltpu.sync_copy(reacc.at[pl.ds(0, NP // 8)],
                    comb.at[hg_own, pl.ds(col0, NP // 8)])
    plsc.subcore_barrier()
    pltpu.sync_copy(comb, dn_t)

    def blockB(b, carry):
        off = base + b * EB_SM
        pltpu.sync_copy(dst_hbm.at[pl.ds(off, EB_SM)], dstb)
        for hg in range(2):
            pltpu.sync_copy(ex_hbm.at[pl.ds((2 * c + hg) * E + off, EB_SM)],
                            exb.at[hg])

        def inner(i, cy):
            s0, q0, s1, q1 = cy
            dstv = dstb[pl.ds(i * 16, 16)]
            outs = []
            for hg in range(2):
                hgv = jnp.full((16,), hg, _i32)
                dnv = plsc.load_gather(dn_t, [hgv, dstv])
                ex = exb[hg, pl.ds(i * 16, 16)]
                a = ex / (dnv + 1e-9)
                alb[hg, pl.ds(i * 16, 16)] = a
                outs.append(a)
            a0, a1 = outs
            return (s0 + a0, q0 + a0 * a0, s1 + a1, q1 + a1 * a1)
        carry = lax.fori_loop(0, EB_SM // 16, inner, carry)
        for hg in range(2):
            pltpu.sync_copy(alb.at[hg],
                            alpha_hbm.at[pl.ds((2 * c + hg) * E + off,
                                               EB_SM)])
        return carry

    z = jnp.zeros((16,), _f32)
    s0, q0, s1, q1 = lax.fori_loop(0, ECHUNK // EB_SM, blockB, (z, z, z, z))
    statsb[pl.ds(0, 16)] = s0
    statsb[pl.ds(16, 16)] = q0
    statsb[pl.ds(32, 16)] = s1
    statsb[pl.ds(48, 16)] = q1
    pltpu.sync_copy(statsb, stats_hbm.at[pl.ds((c * NSUB + sid) * 64, 64)])


def _sc_softmax(src, dst, s_flat, d_flat, el_flat):
    mesh = plsc.VectorSubcoreMesh(core_axis_name="c", subcore_axis_name="s")
    return pl.kernel(
        _softmax_kernel,
        out_type=(
            jax.ShapeDtypeStruct((H * E,), _f32),
            jax.ShapeDtypeStruct((NCORE * NSUB * 64,), _f32),
            jax.ShapeDtypeStruct((H * E,), _f32),
        ),
        mesh=mesh,
        scratch_types=[
            pltpu.VMEM((2, NP), _f32),
            pltpu.VMEM((2, NP), _f32),
            pltpu.VMEM((2, NP), _f32),
            pltpu.VMEM((EB_SM,), _i32),
            pltpu.VMEM((EB_SM,), _i32),
            pltpu.VMEM((2, EB_SM), _f32),
            pltpu.VMEM((2, EB_SM), _f32),
            pltpu.VMEM((2, EB_SM), _f32),
            pltpu.VMEM((64,), _f32),
            pltpu.VMEM_SHARED((NSUB, 2, NP), _f32),
            pltpu.VMEM_SHARED((2, NP), _f32),
        ],
        compiler_params=_SC_PARAMS,
    )(src, dst, s_flat, d_flat, el_flat)


def _agg_node_kernel(src_hbm, dst_hbm, alpha_hbm, hflat_hbm,
                     nacc_hbm,
                     srcb, dstb, idxb, alb, rows, zb, accS, sem):
    c = lax.axis_index("c")
    sid = lax.axis_index("s")
    zero16 = jnp.zeros((16,), _f32)
    row0 = sid * ROWS_T

    for r in range(16):
        for j in range(D // 16):
            zb[r, pl.ds(j * 16, 16)] = zero16

    for hg in range(2):
        for k in range(ROWS_T // 16):
            pltpu.sync_copy(zb, accS.at[pl.ds(row0 + k * 16, 16)])
        plsc.subcore_barrier()
        h_off = (2 * c + hg) * NP

        def blockN(b, _, hg=hg, h_off=h_off):
            eoff = sid * ECHUNK + b * EB_AG
            pltpu.sync_copy(src_hbm.at[pl.ds(eoff, EB_AG)], srcb)
            pltpu.sync_copy(dst_hbm.at[pl.ds(eoff, EB_AG)], dstb)
            pltpu.sync_copy(alpha_hbm.at[pl.ds((2 * c + hg) * E + eoff,
                                               EB_AG)], alb)

            def mkidx(i, _):
                idxb[pl.ds(i * 16, 16)] = srcb[pl.ds(i * 16, 16)] + h_off
                return 0
            lax.fori_loop(0, EB_AG // 16, mkidx, 0)
            pltpu.async_copy(hflat_hbm.at[idxb], rows, sem).wait()

            def scale(r, _):
                rv = jnp.full((16,), r, _i32)
                ab = plsc.load_gather(alb, [rv])
                for j in range(D // 16):
                    rows[r, pl.ds(j * 16, 16)] = rows[r, pl.ds(j * 16, 16)] * ab
                return 0
            lax.fori_loop(0, EB_AG, scale, 0)
            pltpu.sync_copy(rows, accS.at[dstb], add=True)
            return 0
        lax.fori_loop(0, ECHUNK // EB_AG, blockN, 0)
        plsc.subcore_barrier()
        pltpu.sync_copy(accS.at[pl.ds(row0, ROWS_T)],
                        nacc_hbm.at[pl.ds((2 * c + hg) * NP + row0, ROWS_T)])


def _sc_agg_node(src, dst, alpha, hflat):
    mesh = plsc.VectorSubcoreMesh(core_axis_name="c", subcore_axis_name="s")
    return pl.kernel(
        _agg_node_kernel,
        out_type=jax.ShapeDtypeStruct((NCORE * 2 * NP, D), _f32),
        mesh=mesh,
        scratch_types=[
            pltpu.VMEM((EB_AG,), _i32),
            pltpu.VMEM((EB_AG,), _i32),
            pltpu.VMEM((EB_AG,), _i32),
            pltpu.VMEM((EB_AG,), _f32),
            pltpu.VMEM((EB_AG, D), _f32),
            pltpu.VMEM((16, D), _f32),
            pltpu.VMEM_SHARED((NP, D), _f32),
            pltpu.SemaphoreType.DMA,
        ],
        compiler_params=_SC_PARAMS,
    )(src, dst, alpha, hflat)


def _agg_edge_kernel(dst_hbm, alpha_hbm, ef_hbm,
                     eacc_hbm,
                     dstb, alb4, efb, erows, zbe, eaccS):
    c = lax.axis_index("c")
    sid = lax.axis_index("s")
    zero16 = jnp.zeros((16,), _f32)
    row0 = sid * ROWS_T
    wid = sid * NCORE + c
    ebase = wid * EW

    for r in range(16):
        for j in range(H * DE // 16):
            zbe[r, pl.ds(j * 16, 16)] = zero16
    for k in range(ROWS_T // 16):
        pltpu.sync_copy(zbe, eaccS.at[pl.ds(row0 + k * 16, 16)])
    plsc.subcore_barrier()

    def blockE(b, _):
        eoff = ebase + b * EB_EG
        pltpu.sync_copy(dst_hbm.at[pl.ds(eoff, EB_EG)], dstb)
        pltpu.sync_copy(ef_hbm.at[pl.ds(eoff, EB_EG)], efb)
        for h in range(H):
            pltpu.sync_copy(alpha_hbm.at[pl.ds(h * E + eoff, EB_EG)],
                            alb4.at[h])

        def scale(r, _):
            efv = efb[r]
            rv = jnp.full((16,), r, _i32)
            for h in range(H):
                ab = plsc.load_gather(alb4.at[h], [rv])
                erows[r, pl.ds(h * 16, 16)] = efv * ab
            return 0
        lax.fori_loop(0, EB_EG, scale, 0)
        pltpu.sync_copy(erows, eaccS.at[dstb], add=True)
        return 0
    lax.fori_loop(0, EW // EB_EG, blockE, 0)
    plsc.subcore_barrier()
    pltpu.sync_copy(eaccS.at[pl.ds(row0, ROWS_T)],
                    eacc_hbm.at[pl.ds(c * NP + row0, ROWS_T)])


def _sc_agg_edge(dst, alpha, edge_fts):
    mesh = plsc.VectorSubcoreMesh(core_axis_name="c", subcore_axis_name="s")
    return pl.kernel(
        _agg_edge_kernel,
        out_type=jax.ShapeDtypeStruct((NCORE * NP, H * DE), _f32),
        mesh=mesh,
        scratch_types=[
            pltpu.VMEM((EB_EG,), _i32),
            pltpu.VMEM((H, EB_EG), _f32),
            pltpu.VMEM((EB_EG, DE), _f32),
            pltpu.VMEM((EB_EG, H * DE), _f32),
            pltpu.VMEM((16, H * DE), _f32),
            pltpu.VMEM_SHARED((NP, H * DE), _f32),
        ],
        compiler_params=_SC_PARAMS,
    )(dst, alpha, edge_fts)


_BLKO = 1000


def _epilogue_body(nacc_ref, eacc_ref, we_ref, w_ref, out_ref):
    parts = []
    for h in range(H):
        parts.append(nacc_ref[h] * w_ref[h])
    e = eacc_ref[0] + eacc_ref[1]
    for h in range(H):
        eh = jnp.dot(e[:, h * DE:(h + 1) * DE], we_ref[h],
                     preferred_element_type=_f32)
        parts.append(eh * w_ref[h])
    out_ref[...] = jnp.concatenate(parts, axis=1)


def _tc_epilogue(nacc4, eacc, We, wv):
    return pl.pallas_call(
        _epilogue_body,
        grid=(N // _BLKO,),
        in_specs=[
            pl.BlockSpec((H, _BLKO, D), lambda i: (0, i, 0)),
            pl.BlockSpec((NCORE, _BLKO, H * DE), lambda i: (0, i, 0)),
            pl.BlockSpec((H, DE, DE), lambda i: (0, 0, 0)),
            pl.BlockSpec(memory_space=pltpu.SMEM),
        ],
        out_specs=pl.BlockSpec((_BLKO, H * (D + DE)), lambda i: (i, 0)),
        out_shape=jax.ShapeDtypeStruct((N, H * (D + DE)), _f32),
    )(nacc4, eacc, We, wv)


@jax.jit
def kernel(node_fts, edge_fts, edges, Wn, We, a_src, a_dst, a_edge):
    src = edges[0]
    dst = edges[1]

    h_tbl, s_tbl, d_tbl, el = _tc_prep(node_fts, edge_fts, Wn, We,
                                       a_edge, a_src, a_dst)
    s_flat = s_tbl.reshape(H * NP)
    d_flat = d_tbl.reshape(H * NP)
    el_flat = el.reshape(H * E)

    alpha, stats, _ = _sc_softmax(src, dst, s_flat, d_flat, el_flat)

    hflat = h_tbl.reshape(H * NP, D)
    nacc = _sc_agg_node(src, dst, alpha, hflat)
    eacc = _sc_agg_edge(dst, alpha, edge_fts)

    stats = stats.reshape(NCORE, NSUB, 4, 16)
    st = stats.sum(axis=(1, 3))
    sums = st[:, ::2].reshape(H)
    sqs = st[:, 1::2].reshape(H)
    ef32 = jnp.float32(E)
    var = sqs / ef32 - (sums / ef32) ** 2
    wv = jnp.exp(jnp.clip(var, -2.0, 2.0))
    wv = wv / jnp.sum(wv)

    nacc4 = nacc.reshape(H, NP, D)
    eacc2 = eacc.reshape(NCORE, NP, H * DE)
    return _tc_epilogue(nacc4, eacc2, We, wv)


# speedup vs baseline: 12.9016x; 1.1571x over previous
import jax
import jax.numpy as jnp
from jax import lax
from jax.experimental import pallas as pl
from jax.experimental.pallas import tpu as pltpu
from jax.experimental.pallas import tpu_sc as plsc

N = 10000
E = 320000
D = 128
DE = 16
H = 4
NP = 10240
NSUB = 16
NCORE = 2
ECHUNK = E // NSUB
EW = E // (NSUB * NCORE)
EB_SM = 2000
EB_AG = 160
EB_EG = 400
ROWS_T = NP // NSUB

_f32 = jnp.float32
_i32 = jnp.int32

_SC_PARAMS = pltpu.CompilerParams(use_tc_tiling_on_sc=False,
                                  needs_layout_passes=False)


def _prep_nodes_body(x_ref, wn_ref, asrc_ref, adst_ref, h_ref, s_ref, d_ref):
    x = x_ref[...]
    for h in range(H):
        hh = jnp.dot(x, wn_ref[h], preferred_element_type=_f32)
        h_ref[h] = hh
        s_ref[h] = jnp.dot(hh, asrc_ref[h], preferred_element_type=_f32)
        d_ref[h] = jnp.dot(hh, adst_ref[h], preferred_element_type=_f32)


def _prep_edges_body(ef_ref, we_ref, ae_ref, el_ref):
    ef = ef_ref[...]
    for h in range(H):
        q = jnp.dot(we_ref[h], ae_ref[h], preferred_element_type=_f32)
        el_ref[h] = jnp.dot(ef, q, preferred_element_type=_f32)


_BLKN = 1024
_BLKE = 16000


def _tc_prep(node_fts, edge_fts, Wn, We, a_edge, a_src, a_dst):
    h_tbl, s_tbl, d_tbl = pl.pallas_call(
        _prep_nodes_body,
        grid=(NP // _BLKN,),
        in_specs=[
            pl.BlockSpec((_BLKN, D), lambda i: (i, 0)),
            pl.BlockSpec((H, D, D), lambda i: (0, 0, 0)),
            pl.BlockSpec((H, D), lambda i: (0, 0)),
            pl.BlockSpec((H, D), lambda i: (0, 0)),
        ],
        out_specs=[
            pl.BlockSpec((H, _BLKN, D), lambda i: (0, i, 0)),
            pl.BlockSpec((H, _BLKN), lambda i: (0, i)),
            pl.BlockSpec((H, _BLKN), lambda i: (0, i)),
        ],
        out_shape=[
            jax.ShapeDtypeStruct((H, NP, D), _f32),
            jax.ShapeDtypeStruct((H, NP), _f32),
            jax.ShapeDtypeStruct((H, NP), _f32),
        ],
    )(node_fts, Wn, a_src, a_dst)

    el = pl.pallas_call(
        _prep_edges_body,
        grid=(E // _BLKE,),
        in_specs=[
            pl.BlockSpec((_BLKE, DE), lambda i: (i, 0)),
            pl.BlockSpec((H, DE, DE), lambda i: (0, 0, 0)),
            pl.BlockSpec((H, DE), lambda i: (0, 0)),
        ],
        out_specs=pl.BlockSpec((H, _BLKE), lambda i: (0, i)),
        out_shape=jax.ShapeDtypeStruct((H, E), _f32),
    )(edge_fts, We, a_edge)
    return h_tbl, s_tbl, d_tbl, el


def _softmax_kernel(src_hbm, dst_hbm, s_hbm, d_hbm, el_hbm,
                    alpha_hbm, stats_hbm, ex_hbm,
                    s_t, d_t, dn_t, srcb, dstb, elb, exb, alb, statsb,
                    slots, comb):
    c = lax.axis_index("c")
    sid = lax.axis_index("s")
    base = sid * ECHUNK
    zero16 = jnp.zeros((16,), _f32)

    for hg in range(2):
        pltpu.sync_copy(s_hbm.at[pl.ds(c * (2 * NP) + hg * NP, NP)],
                        s_t.at[hg])
        pltpu.sync_copy(d_hbm.at[pl.ds(c * (2 * NP) + hg * NP, NP)],
                        d_t.at[hg])

        def zbody(i, _, hg=hg):
            dn_t[hg, pl.ds(i * 16, 16)] = zero16
            return 0
        lax.fori_loop(0, NP // 16, zbody, 0)

    def blockA(b, _):
        off = base + b * EB_SM
        pltpu.sync_copy(src_hbm.at[pl.ds(off, EB_SM)], srcb)
        pltpu.sync_copy(dst_hbm.at[pl.ds(off, EB_SM)], dstb)
        for hg in range(2):
            pltpu.sync_copy(el_hbm.at[pl.ds((2 * c + hg) * E + off, EB_SM)],
                            elb.at[hg])

        def inner(i, _):
            srcv = srcb[pl.ds(i * 16, 16)]
            dstv = dstb[pl.ds(i * 16, 16)]
            for hg in range(2):
                hgv = jnp.full((16,), hg, _i32)
                sv = plsc.load_gather(s_t, [hgv, srcv])
                dv = plsc.load_gather(d_t, [hgv, dstv])
                l = sv + dv + elb[hg, pl.ds(i * 16, 16)]
                l = jnp.where(l >= 0.0, l, 0.2 * l)
                ex = jnp.exp(l)
                exb[hg, pl.ds(i * 16, 16)] = ex
                plsc.addupdate_scatter(dn_t, [hgv, dstv], ex)
            return 0
        lax.fori_loop(0, EB_SM // 16, inner, 0)
        for hg in range(2):
            pltpu.sync_copy(exb.at[hg],
                            ex_hbm.at[pl.ds((2 * c + hg) * E + off, EB_SM)])
        return 0
    lax.fori_loop(0, ECHUNK // EB_SM, blockA, 0)

    pltpu.sync_copy(dn_t, slots.at[sid])
    plsc.subcore_barrier()
    hg_own = sid // 8
    col0 = (sid % 8) * (NP // 8)
    redbuf = alb.at[0]
    reacc = alb.at[1]

    def zr(i, _):
        reacc[pl.ds(i * 16, 16)] = zero16
        return 0
    lax.fori_loop(0, (NP // 8) // 16, zr, 0)
    for j in range(NSUB):
        pltpu.sync_copy(slots.at[j, hg_own, pl.ds(col0, NP // 8)],
                        redbuf.at[pl.ds(0, NP // 8)])

        def addr(i, _):
            reacc[pl.ds(i * 16, 16)] = (reacc[pl.ds(i * 16, 16)]
                                        + redbuf[pl.ds(i * 16, 16)])
            return 0
        lax.fori_loop(0, (NP // 8) // 16, addr, 0)
    pltpu.sync_copy(reacc.at[pl.ds(0, NP // 8)],
                    comb.at[hg_own, pl.ds(col0, NP // 8)])
    plsc.subcore_barrier()
    pltpu.sync_copy(comb, dn_t)

    def blockB(b, carry):
        off = base + b * EB_SM
        pltpu.sync_copy(dst_hbm.at[pl.ds(off, EB_SM)], dstb)
        for hg in range(2):
            pltpu.sync_copy(ex_hbm.at[pl.ds((2 * c + hg) * E + off, EB_SM)],
                            exb.at[hg])

        def inner(i, cy):
            s0, q0, s1, q1 = cy
            dstv = dstb[pl.ds(i * 16, 16)]
            outs = []
            for hg in range(2):
                hgv = jnp.full((16,), hg, _i32)
                dnv = plsc.load_gather(dn_t, [hgv, dstv])
                ex = exb[hg, pl.ds(i * 16, 16)]
                a = ex / (dnv + 1e-9)
                alb[hg, pl.ds(i * 16, 16)] = a
                outs.append(a)
            a0, a1 = outs
            return (s0 + a0, q0 + a0 * a0, s1 + a1, q1 + a1 * a1)
        carry = lax.fori_loop(0, EB_SM // 16, inner, carry)
        for hg in range(2):
            pltpu.sync_copy(alb.at[hg],
                            alpha_hbm.at[pl.ds((2 * c + hg) * E + off,
                                               EB_SM)])
        return carry

    z = jnp.zeros((16,), _f32)
    s0, q0, s1, q1 = lax.fori_loop(0, ECHUNK // EB_SM, blockB, (z, z, z, z))
    statsb[pl.ds(0, 16)] = s0
    statsb[pl.ds(16, 16)] = q0
    statsb[pl.ds(32, 16)] = s1
    statsb[pl.ds(48, 16)] = q1
    pltpu.sync_copy(statsb, stats_hbm.at[pl.ds((c * NSUB + sid) * 64, 64)])


def _sc_softmax(src, dst, s_flat, d_flat, el_flat):
    mesh = plsc.VectorSubcoreMesh(core_axis_name="c", subcore_axis_name="s")
    return pl.kernel(
        _softmax_kernel,
        out_type=(
            jax.ShapeDtypeStruct((H * E,), _f32),
            jax.ShapeDtypeStruct((NCORE * NSUB * 64,), _f32),
            jax.ShapeDtypeStruct((H * E,), _f32),
        ),
        mesh=mesh,
        scratch_types=[
            pltpu.VMEM((2, NP), _f32),
            pltpu.VMEM((2, NP), _f32),
            pltpu.VMEM((2, NP), _f32),
            pltpu.VMEM((EB_SM,), _i32),
            pltpu.VMEM((EB_SM,), _i32),
            pltpu.VMEM((2, EB_SM), _f32),
            pltpu.VMEM((2, EB_SM), _f32),
            pltpu.VMEM((2, EB_SM), _f32),
            pltpu.VMEM((64,), _f32),
            pltpu.VMEM_SHARED((NSUB, 2, NP), _f32),
            pltpu.VMEM_SHARED((2, NP), _f32),
        ],
        compiler_params=_SC_PARAMS,
    )(src, dst, s_flat, d_flat, el_flat)


def _agg_node_kernel(src_hbm, dst_hbm, alpha_hbm, hflat_hbm,
                     nacc_hbm,
                     idx0, idx1, dstb0, dstb1, alb0, alb1,
                     rows0, rows1, zb, accS, sem0, sem1):
    c = lax.axis_index("c")
    sid = lax.axis_index("s")
    zero16 = jnp.zeros((16,), _f32)
    row0 = sid * ROWS_T
    ebase = sid * ECHUNK
    nblk = ECHUNK // EB_AG

    for r in range(16):
        for j in range(D // 16):
            zb[r, pl.ds(j * 16, 16)] = zero16

    for hg in range(2):
        for k in range(ROWS_T // 16):
            pltpu.sync_copy(zb, accS.at[pl.ds(row0 + k * 16, 16)])
        plsc.subcore_barrier()
        h_off = (2 * c + hg) * NP
        a_base = (2 * c + hg) * E + ebase

        def fetch(idx, dstb, alb, blk, h_off=h_off, a_base=a_base):
            off = ebase + blk * EB_AG
            pltpu.sync_copy(src_hbm.at[pl.ds(off, EB_AG)], idx)
            pltpu.sync_copy(dst_hbm.at[pl.ds(off, EB_AG)], dstb)
            pltpu.sync_copy(alpha_hbm.at[pl.ds(a_base + blk * EB_AG, EB_AG)],
                            alb)

            def go(i, _):
                idx[pl.ds(i * 16, 16)] = idx[pl.ds(i * 16, 16)] + h_off
                return 0
            lax.fori_loop(0, EB_AG // 16, go, 0)

        def process(rows, dstb, alb):
            def scale(r, _):
                rv = jnp.full((16,), r, _i32)
                ab = plsc.load_gather(alb, [rv])
                for j in range(D // 16):
                    rows[r, pl.ds(j * 16, 16)] = (rows[r, pl.ds(j * 16, 16)]
                                                  * ab)
                return 0
            lax.fori_loop(0, EB_AG, scale, 0)
            pltpu.sync_copy(rows, accS.at[dstb], add=True)

        fetch(idx0, dstb0, alb0, 0)
        pltpu.async_copy(hflat_hbm.at[idx0], rows0, sem0)

        def pair(i, _):
            fetch(idx1, dstb1, alb1, 2 * i + 1)
            pltpu.async_copy(hflat_hbm.at[idx1], rows1, sem1)
            pltpu.make_async_copy(hflat_hbm.at[idx0], rows0, sem0).wait()
            process(rows0, dstb0, alb0)
            fetch(idx0, dstb0, alb0, 2 * i + 2)
            pltpu.async_copy(hflat_hbm.at[idx0], rows0, sem0)
            pltpu.make_async_copy(hflat_hbm.at[idx1], rows1, sem1).wait()
            process(rows1, dstb1, alb1)
            return 0
        lax.fori_loop(0, (nblk - 1) // 2, pair, 0)
        pltpu.make_async_copy(hflat_hbm.at[idx0], rows0, sem0).wait()
        process(rows0, dstb0, alb0)

        plsc.subcore_barrier()
        pltpu.sync_copy(accS.at[pl.ds(row0, ROWS_T)],
                        nacc_hbm.at[pl.ds((2 * c + hg) * NP + row0, ROWS_T)])


def _sc_agg_node(src, dst, alpha, hflat):
    mesh = plsc.VectorSubcoreMesh(core_axis_name="c", subcore_axis_name="s")
    return pl.kernel(
        _agg_node_kernel,
        out_type=jax.ShapeDtypeStruct((NCORE * 2 * NP, D), _f32),
        mesh=mesh,
        scratch_types=[
            pltpu.VMEM((EB_AG,), _i32),
            pltpu.VMEM((EB_AG,), _i32),
            pltpu.VMEM((EB_AG,), _i32),
            pltpu.VMEM((EB_AG,), _i32),
            pltpu.VMEM((EB_AG,), _f32),
            pltpu.VMEM((EB_AG,), _f32),
            pltpu.VMEM((EB_AG, D), _f32),
            pltpu.VMEM((EB_AG, D), _f32),
            pltpu.VMEM((16, D), _f32),
            pltpu.VMEM_SHARED((NP, D), _f32),
            pltpu.SemaphoreType.DMA,
            pltpu.SemaphoreType.DMA,
        ],
        compiler_params=_SC_PARAMS,
    )(src, dst, alpha, hflat)


def _agg_edge_kernel(dst_hbm, alpha_hbm, ef_hbm,
                     eacc_hbm,
                     dstb, alb4, efb, erows, zbe, eaccS):
    c = lax.axis_index("c")
    sid = lax.axis_index("s")
    zero16 = jnp.zeros((16,), _f32)
    row0 = sid * ROWS_T
    wid = sid * NCORE + c
    ebase = wid * EW

    for r in range(16):
        for j in range(H * DE // 16):
            zbe[r, pl.ds(j * 16, 16)] = zero16
    for k in range(ROWS_T // 16):
        pltpu.sync_copy(zbe, eaccS.at[pl.ds(row0 + k * 16, 16)])
    plsc.subcore_barrier()

    def blockE(b, _):
        eoff = ebase + b * EB_EG
        pltpu.sync_copy(dst_hbm.at[pl.ds(eoff, EB_EG)], dstb)
        pltpu.sync_copy(ef_hbm.at[pl.ds(eoff, EB_EG)], efb)
        for h in range(H):
            pltpu.sync_copy(alpha_hbm.at[pl.ds(h * E + eoff, EB_EG)],
                            alb4.at[h])

        def scale(r, _):
            efv = efb[r]
            rv = jnp.full((16,), r, _i32)
            for h in range(H):
                ab = plsc.load_gather(alb4.at[h], [rv])
                erows[r, pl.ds(h * 16, 16)] = efv * ab
            return 0
        lax.fori_loop(0, EB_EG, scale, 0)
        pltpu.sync_copy(erows, eaccS.at[dstb], add=True)
        return 0
    lax.fori_loop(0, EW // EB_EG, blockE, 0)
    plsc.subcore_barrier()
    pltpu.sync_copy(eaccS.at[pl.ds(row0, ROWS_T)],
                    eacc_hbm.at[pl.ds(c * NP + row0, ROWS_T)])


def _sc_agg_edge(dst, alpha, edge_fts):
    mesh = plsc.VectorSubcoreMesh(core_axis_name="c", subcore_axis_name="s")
    return pl.kernel(
        _agg_edge_kernel,
        out_type=jax.ShapeDtypeStruct((NCORE * NP, H * DE), _f32),
        mesh=mesh,
        scratch_types=[
            pltpu.VMEM((EB_EG,), _i32),
            pltpu.VMEM((H, EB_EG), _f32),
            pltpu.VMEM((EB_EG, DE), _f32),
            pltpu.VMEM((EB_EG, H * DE), _f32),
            pltpu.VMEM((16, H * DE), _f32),
            pltpu.VMEM_SHARED((NP, H * DE), _f32),
        ],
        compiler_params=_SC_PARAMS,
    )(dst, alpha, edge_fts)


_BLKO = 1000


def _epilogue_body(nacc_ref, eacc_ref, we_ref, w_ref, out_ref):
    parts = []
    for h in range(H):
        parts.append(nacc_ref[h] * w_ref[h])
    e = eacc_ref[0] + eacc_ref[1]
    for h in range(H):
        eh = jnp.dot(e[:, h * DE:(h + 1) * DE], we_ref[h],
                     preferred_element_type=_f32)
        parts.append(eh * w_ref[h])
    out_ref[...] = jnp.concatenate(parts, axis=1)


def _tc_epilogue(nacc4, eacc, We, wv):
    return pl.pallas_call(
        _epilogue_body,
        grid=(N // _BLKO,),
        in_specs=[
            pl.BlockSpec((H, _BLKO, D), lambda i: (0, i, 0)),
            pl.BlockSpec((NCORE, _BLKO, H * DE), lambda i: (0, i, 0)),
            pl.BlockSpec((H, DE, DE), lambda i: (0, 0, 0)),
            pl.BlockSpec(memory_space=pltpu.SMEM),
        ],
        out_specs=pl.BlockSpec((_BLKO, H * (D + DE)), lambda i: (i, 0)),
        out_shape=jax.ShapeDtypeStruct((N, H * (D + DE)), _f32),
    )(nacc4, eacc, We, wv)


@jax.jit
def kernel(node_fts, edge_fts, edges, Wn, We, a_src, a_dst, a_edge):
    src = edges[0]
    dst = edges[1]

    h_tbl, s_tbl, d_tbl, el = _tc_prep(node_fts, edge_fts, Wn, We,
                                       a_edge, a_src, a_dst)
    s_flat = s_tbl.reshape(H * NP)
    d_flat = d_tbl.reshape(H * NP)
    el_flat = el.reshape(H * E)

    alpha, stats, _ = _sc_softmax(src, dst, s_flat, d_flat, el_flat)

    hflat = h_tbl.reshape(H * NP, D)
    nacc = _sc_agg_node(src, dst, alpha, hflat)
    eacc = _sc_agg_edge(dst, alpha, edge_fts)

    stats = stats.reshape(NCORE, NSUB, 4, 16)
    st = stats.sum(axis=(1, 3))
    sums = st[:, ::2].reshape(H)
    sqs = st[:, 1::2].reshape(H)
    ef32 = jnp.float32(E)
    var = sqs / ef32 - (sums / ef32) ** 2
    wv = jnp.exp(jnp.clip(var, -2.0, 2.0))
    wv = wv / jnp.sum(wv)

    nacc4 = nacc.reshape(H, NP, D)
    eacc2 = eacc.reshape(NCORE, NP, H * DE)
    return _tc_epilogue(nacc4, eacc2, We, wv)


# speedup vs baseline: 14.5151x; 1.1251x over previous
import jax
import jax.numpy as jnp
from jax import lax
from jax.experimental import pallas as pl
from jax.experimental.pallas import tpu as pltpu
from jax.experimental.pallas import tpu_sc as plsc

N = 10000
E = 320000
D = 128
DE = 16
H = 4
NP = 10240
NSUB = 16
NCORE = 2
ECHUNK = E // NSUB
EW = E // (NSUB * NCORE)
EB_SM = 2000
EB_AG = 80
ES_AG = 2000
EB_EG = 400
ROWS_T = NP // NSUB

_f32 = jnp.float32
_i32 = jnp.int32

_SC_PARAMS = pltpu.CompilerParams(use_tc_tiling_on_sc=False,
                                  needs_layout_passes=False)


def _prep_nodes_body(x_ref, wn_ref, asrc_ref, adst_ref, h_ref, s_ref, d_ref):
    x = x_ref[...]
    for h in range(H):
        hh = jnp.dot(x, wn_ref[h], preferred_element_type=_f32)
        h_ref[h] = hh
        s_ref[h] = jnp.dot(hh, asrc_ref[h], preferred_element_type=_f32)
        d_ref[h] = jnp.dot(hh, adst_ref[h], preferred_element_type=_f32)


def _prep_edges_body(ef_ref, we_ref, ae_ref, el_ref):
    ef = ef_ref[...]
    for h in range(H):
        q = jnp.dot(we_ref[h], ae_ref[h], preferred_element_type=_f32)
        el_ref[h] = jnp.dot(ef, q, preferred_element_type=_f32)


_BLKN = 1024
_BLKE = 16000


def _tc_prep(node_fts, edge_fts, Wn, We, a_edge, a_src, a_dst):
    h_tbl, s_tbl, d_tbl = pl.pallas_call(
        _prep_nodes_body,
        grid=(NP // _BLKN,),
        in_specs=[
            pl.BlockSpec((_BLKN, D), lambda i: (i, 0)),
            pl.BlockSpec((H, D, D), lambda i: (0, 0, 0)),
            pl.BlockSpec((H, D), lambda i: (0, 0)),
            pl.BlockSpec((H, D), lambda i: (0, 0)),
        ],
        out_specs=[
            pl.BlockSpec((H, _BLKN, D), lambda i: (0, i, 0)),
            pl.BlockSpec((H, _BLKN), lambda i: (0, i)),
            pl.BlockSpec((H, _BLKN), lambda i: (0, i)),
        ],
        out_shape=[
            jax.ShapeDtypeStruct((H, NP, D), _f32),
            jax.ShapeDtypeStruct((H, NP), _f32),
            jax.ShapeDtypeStruct((H, NP), _f32),
        ],
    )(node_fts, Wn, a_src, a_dst)

    el = pl.pallas_call(
        _prep_edges_body,
        grid=(E // _BLKE,),
        in_specs=[
            pl.BlockSpec((_BLKE, DE), lambda i: (i, 0)),
            pl.BlockSpec((H, DE, DE), lambda i: (0, 0, 0)),
            pl.BlockSpec((H, DE), lambda i: (0, 0)),
        ],
        out_specs=pl.BlockSpec((H, _BLKE), lambda i: (0, i)),
        out_shape=jax.ShapeDtypeStruct((H, E), _f32),
    )(edge_fts, We, a_edge)
    return h_tbl, s_tbl, d_tbl, el


def _softmax_kernel(src_hbm, dst_hbm, s_hbm, d_hbm, el_hbm,
                    alpha_hbm, stats_hbm, ex_hbm,
                    s_t, d_t, dn_t, srcb, dstb, elb, exb, alb, statsb,
                    slots, comb):
    c = lax.axis_index("c")
    sid = lax.axis_index("s")
    base = sid * ECHUNK
    zero16 = jnp.zeros((16,), _f32)

    for hg in range(2):
        pltpu.sync_copy(s_hbm.at[pl.ds(c * (2 * NP) + hg * NP, NP)],
                        s_t.at[hg])
        pltpu.sync_copy(d_hbm.at[pl.ds(c * (2 * NP) + hg * NP, NP)],
                        d_t.at[hg])

        def zbody(i, _, hg=hg):
            dn_t[hg, pl.ds(i * 16, 16)] = zero16
            return 0
        lax.fori_loop(0, NP // 16, zbody, 0)

    def blockA(b, _):
        off = base + b * EB_SM
        pltpu.sync_copy(src_hbm.at[pl.ds(off, EB_SM)], srcb)
        pltpu.sync_copy(dst_hbm.at[pl.ds(off, EB_SM)], dstb)
        for hg in range(2):
            pltpu.sync_copy(el_hbm.at[pl.ds((2 * c + hg) * E + off, EB_SM)],
                            elb.at[hg])

        def inner(i, _):
            srcv = srcb[pl.ds(i * 16, 16)]
            dstv = dstb[pl.ds(i * 16, 16)]
            for hg in range(2):
                hgv = jnp.full((16,), hg, _i32)
                sv = plsc.load_gather(s_t, [hgv, srcv])
                dv = plsc.load_gather(d_t, [hgv, dstv])
                l = sv + dv + elb[hg, pl.ds(i * 16, 16)]
                l = jnp.where(l >= 0.0, l, 0.2 * l)
                ex = jnp.exp(l)
                exb[hg, pl.ds(i * 16, 16)] = ex
                plsc.addupdate_scatter(dn_t, [hgv, dstv], ex)
            return 0
        lax.fori_loop(0, EB_SM // 16, inner, 0)
        for hg in range(2):
            pltpu.sync_copy(exb.at[hg],
                            ex_hbm.at[pl.ds((2 * c + hg) * E + off, EB_SM)])
        return 0
    lax.fori_loop(0, ECHUNK // EB_SM, blockA, 0)

    pltpu.sync_copy(dn_t, slots.at[sid])
    plsc.subcore_barrier()
    hg_own = sid // 8
    col0 = (sid % 8) * (NP // 8)
    redbuf = alb.at[0]
    reacc = alb.at[1]

    def zr(i, _):
        reacc[pl.ds(i * 16, 16)] = zero16
        return 0
    lax.fori_loop(0, (NP // 8) // 16, zr, 0)
    for j in range(NSUB):
        pltpu.sync_copy(slots.at[j, hg_own, pl.ds(col0, NP // 8)],
                        redbuf.at[pl.ds(0, NP // 8)])

        def addr(i, _):
            reacc[pl.ds(i * 16, 16)] = (reacc[pl.ds(i * 16, 16)]
                                        + redbuf[pl.ds(i * 16, 16)])
            return 0
        lax.fori_loop(0, (NP // 8) // 16, addr, 0)
    pltpu.sync_copy(reacc.at[pl.ds(0, NP // 8)],
                    comb.at[hg_own, pl.ds(col0, NP // 8)])
    plsc.subcore_barrier()
    pltpu.sync_copy(comb, dn_t)

    def blockB(b, carry):
        off = base + b * EB_SM
        pltpu.sync_copy(dst_hbm.at[pl.ds(off, EB_SM)], dstb)
        for hg in range(2):
            pltpu.sync_copy(ex_hbm.at[pl.ds((2 * c + hg) * E + off, EB_SM)],
                            exb.at[hg])

        def inner(i, cy):
            s0, q0, s1, q1 = cy
            dstv = dstb[pl.ds(i * 16, 16)]
            outs = []
            for hg in range(2):
                hgv = jnp.full((16,), hg, _i32)
                dnv = plsc.load_gather(dn_t, [hgv, dstv])
                ex = exb[hg, pl.ds(i * 16, 16)]
                a = ex / (dnv + 1e-9)
                alb[hg, pl.ds(i * 16, 16)] = a
                outs.append(a)
            a0, a1 = outs
            return (s0 + a0, q0 + a0 * a0, s1 + a1, q1 + a1 * a1)
        carry = lax.fori_loop(0, EB_SM // 16, inner, carry)
        for hg in range(2):
            pltpu.sync_copy(alb.at[hg],
                            alpha_hbm.at[pl.ds((2 * c + hg) * E + off,
                                               EB_SM)])
        return carry

    z = jnp.zeros((16,), _f32)
    s0, q0, s1, q1 = lax.fori_loop(0, ECHUNK // EB_SM, blockB, (z, z, z, z))
    statsb[pl.ds(0, 16)] = s0
    statsb[pl.ds(16, 16)] = q0
    statsb[pl.ds(32, 16)] = s1
    statsb[pl.ds(48, 16)] = q1
    pltpu.sync_copy(statsb, stats_hbm.at[pl.ds((c * NSUB + sid) * 64, 64)])


def _sc_softmax(src, dst, s_flat, d_flat, el_flat):
    mesh = plsc.VectorSubcoreMesh(core_axis_name="c", subcore_axis_name="s")
    return pl.kernel(
        _softmax_kernel,
        out_type=(
            jax.ShapeDtypeStruct((H * E,), _f32),
            jax.ShapeDtypeStruct((NCORE * NSUB * 64,), _f32),
            jax.ShapeDtypeStruct((H * E,), _f32),
        ),
        mesh=mesh,
        scratch_types=[
            pltpu.VMEM((2, NP), _f32),
            pltpu.VMEM((2, NP), _f32),
            pltpu.VMEM((2, NP), _f32),
            pltpu.VMEM((EB_SM,), _i32),
            pltpu.VMEM((EB_SM,), _i32),
            pltpu.VMEM((2, EB_SM), _f32),
            pltpu.VMEM((2, EB_SM), _f32),
            pltpu.VMEM((2, EB_SM), _f32),
            pltpu.VMEM((64,), _f32),
            pltpu.VMEM_SHARED((NSUB, 2, NP), _f32),
            pltpu.VMEM_SHARED((2, NP), _f32),
        ],
        compiler_params=_SC_PARAMS,
    )(src, dst, s_flat, d_flat, el_flat)


def _agg_node_kernel(src_hbm, dst_hbm, alpha_hbm, hflat_hbm,
                     nacc_hbm,
                     srcS, dstS, alS, idx0, idx1, dstb0, dstb1,
                     rows0, rows1, zb, accS, sem0, sem1):
    c = lax.axis_index("c")
    sid = lax.axis_index("s")
    zero16 = jnp.zeros((16,), _f32)
    row0 = sid * ROWS_T
    ebase = sid * ECHUNK
    nsup = ECHUNK // ES_AG
    nblk = ES_AG // EB_AG

    for r in range(16):
        for j in range(D // 16):
            zb[r, pl.ds(j * 16, 16)] = zero16

    for hg in range(2):
        for k in range(ROWS_T // 16):
            pltpu.sync_copy(zb, accS.at[pl.ds(row0 + k * 16, 16)])
        plsc.subcore_barrier()
        h_off = (2 * c + hg) * NP
        a_base = (2 * c + hg) * E + ebase

        def sup_body(sp, _, h_off=h_off, a_base=a_base):
            soff = ebase + sp * ES_AG
            pltpu.sync_copy(src_hbm.at[pl.ds(soff, ES_AG)], srcS)
            pltpu.sync_copy(dst_hbm.at[pl.ds(soff, ES_AG)], dstS)
            pltpu.sync_copy(alpha_hbm.at[pl.ds(a_base + sp * ES_AG, ES_AG)],
                            alS)

            def prep(idx, dstb, blk):
                off = blk * EB_AG

                def go(i, _):
                    idx[pl.ds(i * 16, 16)] = (srcS[pl.ds(off + i * 16, 16)]
                                              + h_off)
                    dstb[pl.ds(i * 16, 16)] = dstS[pl.ds(off + i * 16, 16)]
                    return 0
                lax.fori_loop(0, EB_AG // 16, go, 0)

            def process(rows, dstb, blk):
                off = blk * EB_AG

                def scale(r, _):
                    rv = jnp.full((16,), off + r, _i32)
                    ab = plsc.load_gather(alS, [rv])
                    for j in range(D // 16):
                        rows[r, pl.ds(j * 16, 16)] = (
                            rows[r, pl.ds(j * 16, 16)] * ab)
                    return 0
                lax.fori_loop(0, EB_AG, scale, 0)
                pltpu.sync_copy(rows, accS.at[dstb], add=True)

            prep(idx0, dstb0, 0)
            pltpu.async_copy(hflat_hbm.at[idx0], rows0, sem0)

            def pr(i, _):
                prep(idx1, dstb1, 2 * i + 1)
                pltpu.async_copy(hflat_hbm.at[idx1], rows1, sem1)
                pltpu.make_async_copy(hflat_hbm.at[idx0], rows0, sem0).wait()
                process(rows0, dstb0, 2 * i)
                prep(idx0, dstb0, 2 * i + 2)
                pltpu.async_copy(hflat_hbm.at[idx0], rows0, sem0)
                pltpu.make_async_copy(hflat_hbm.at[idx1], rows1, sem1).wait()
                process(rows1, dstb1, 2 * i + 1)
                return 0
            lax.fori_loop(0, (nblk - 1) // 2, pr, 0)
            pltpu.make_async_copy(hflat_hbm.at[idx0], rows0, sem0).wait()
            process(rows0, dstb0, nblk - 1)
            return 0
        lax.fori_loop(0, nsup, sup_body, 0)

        plsc.subcore_barrier()
        pltpu.sync_copy(accS.at[pl.ds(row0, ROWS_T)],
                        nacc_hbm.at[pl.ds((2 * c + hg) * NP + row0, ROWS_T)])


def _sc_agg_node(src, dst, alpha, hflat):
    mesh = plsc.VectorSubcoreMesh(core_axis_name="c", subcore_axis_name="s")
    return pl.kernel(
        _agg_node_kernel,
        out_type=jax.ShapeDtypeStruct((NCORE * 2 * NP, D), _f32),
        mesh=mesh,
        scratch_types=[
            pltpu.VMEM((ES_AG,), _i32),
            pltpu.VMEM((ES_AG,), _i32),
            pltpu.VMEM((ES_AG,), _f32),
            pltpu.VMEM((EB_AG,), _i32),
            pltpu.VMEM((EB_AG,), _i32),
            pltpu.VMEM((EB_AG,), _i32),
            pltpu.VMEM((EB_AG,), _i32),
            pltpu.VMEM((EB_AG, D), _f32),
            pltpu.VMEM((EB_AG, D), _f32),
            pltpu.VMEM((16, D), _f32),
            pltpu.VMEM_SHARED((NP, D), _f32),
            pltpu.SemaphoreType.DMA,
            pltpu.SemaphoreType.DMA,
        ],
        compiler_params=_SC_PARAMS,
    )(src, dst, alpha, hflat)


def _agg_edge_kernel(dst_hbm, alpha_hbm, ef_hbm,
                     eacc_hbm,
                     dstb, alb4, efb, erows, zbe, eaccS):
    c = lax.axis_index("c")
    sid = lax.axis_index("s")
    zero16 = jnp.zeros((16,), _f32)
    row0 = sid * ROWS_T
    wid = sid * NCORE + c
    ebase = wid * EW

    for r in range(16):
        for j in range(H * DE // 16):
            zbe[r, pl.ds(j * 16, 16)] = zero16
    for k in range(ROWS_T // 16):
        pltpu.sync_copy(zbe, eaccS.at[pl.ds(row0 + k * 16, 16)])
    plsc.subcore_barrier()

    def blockE(b, _):
        eoff = ebase + b * EB_EG
        pltpu.sync_copy(dst_hbm.at[pl.ds(eoff, EB_EG)], dstb)
        pltpu.sync_copy(ef_hbm.at[pl.ds(eoff, EB_EG)], efb)
        for h in range(H):
            pltpu.sync_copy(alpha_hbm.at[pl.ds(h * E + eoff, EB_EG)],
                            alb4.at[h])

        def scale(r, _):
            efv = efb[r]
            rv = jnp.full((16,), r, _i32)
            for h in range(H):
                ab = plsc.load_gather(alb4.at[h], [rv])
                erows[r, pl.ds(h * 16, 16)] = efv * ab
            return 0
        lax.fori_loop(0, EB_EG, scale, 0)
        pltpu.sync_copy(erows, eaccS.at[dstb], add=True)
        return 0
    lax.fori_loop(0, EW // EB_EG, blockE, 0)
    plsc.subcore_barrier()
    pltpu.sync_copy(eaccS.at[pl.ds(row0, ROWS_T)],
                    eacc_hbm.at[pl.ds(c * NP + row0, ROWS_T)])


def _sc_agg_edge(dst, alpha, edge_fts):
    mesh = plsc.VectorSubcoreMesh(core_axis_name="c", subcore_axis_name="s")
    return pl.kernel(
        _agg_edge_kernel,
        out_type=jax.ShapeDtypeStruct((NCORE * NP, H * DE), _f32),
        mesh=mesh,
        scratch_types=[
            pltpu.VMEM((EB_EG,), _i32),
            pltpu.VMEM((H, EB_EG), _f32),
            pltpu.VMEM((EB_EG, DE), _f32),
            pltpu.VMEM((EB_EG, H * DE), _f32),
            pltpu.VMEM((16, H * DE), _f32),
            pltpu.VMEM_SHARED((NP, H * DE), _f32),
        ],
        compiler_params=_SC_PARAMS,
    )(dst, alpha, edge_fts)


_BLKO = 1000


def _epilogue_body(nacc_ref, eacc_ref, we_ref, w_ref, out_ref):
    parts = []
    for h in range(H):
        parts.append(nacc_ref[h] * w_ref[h])
    e = eacc_ref[0] + eacc_ref[1]
    for h in range(H):
        eh = jnp.dot(e[:, h * DE:(h + 1) * DE], we_ref[h],
                     preferred_element_type=_f32)
        parts.append(eh * w_ref[h])
    out_ref[...] = jnp.concatenate(parts, axis=1)


def _tc_epilogue(nacc4, eacc, We, wv):
    return pl.pallas_call(
        _epilogue_body,
        grid=(N // _BLKO,),
        in_specs=[
            pl.BlockSpec((H, _BLKO, D), lambda i: (0, i, 0)),
            pl.BlockSpec((NCORE, _BLKO, H * DE), lambda i: (0, i, 0)),
            pl.BlockSpec((H, DE, DE), lambda i: (0, 0, 0)),
            pl.BlockSpec(memory_space=pltpu.SMEM),
        ],
        out_specs=pl.BlockSpec((_BLKO, H * (D + DE)), lambda i: (i, 0)),
        out_shape=jax.ShapeDtypeStruct((N, H * (D + DE)), _f32),
    )(nacc4, eacc, We, wv)


@jax.jit
def kernel(node_fts, edge_fts, edges, Wn, We, a_src, a_dst, a_edge):
    src = edges[0]
    dst = edges[1]

    h_tbl, s_tbl, d_tbl, el = _tc_prep(node_fts, edge_fts, Wn, We,
                                       a_edge, a_src, a_dst)
    s_flat = s_tbl.reshape(H * NP)
    d_flat = d_tbl.reshape(H * NP)
    el_flat = el.reshape(H * E)

    alpha, stats, _ = _sc_softmax(src, dst, s_flat, d_flat, el_flat)

    hflat = h_tbl.reshape(H * NP, D)
    nacc = _sc_agg_node(src, dst, alpha, hflat)
    eacc = _sc_agg_edge(dst, alpha, edge_fts)

    stats = stats.reshape(NCORE, NSUB, 4, 16)
    st = stats.sum(axis=(1, 3))
    sums = st[:, ::2].reshape(H)
    sqs = st[:, 1::2].reshape(H)
    ef32 = jnp.float32(E)
    var = sqs / ef32 - (sums / ef32) ** 2
    wv = jnp.exp(jnp.clip(var, -2.0, 2.0))
    wv = wv / jnp.sum(wv)

    nacc4 = nacc.reshape(H, NP, D)
    eacc2 = eacc.reshape(NCORE, NP, H * DE)
    return _tc_epilogue(nacc4, eacc2, We, wv)
